# baseline (speedup 1.0000x reference)
import functools

import jax
import jax.numpy as jnp
from jax import lax
from jax.experimental import pallas as pl
from jax.experimental.pallas import tpu as pltpu
from jax.experimental.pallas import tpu_sc as plsc

N_I = 10000
N_D = 10000
E = 320000
D = 128

NC = 2
NS = 16
NW = NC * NS
EP = 327680
C = 128
NBLK = 10
IB = 8
AR = 10112
RPT = AR // NS
NBUF = 2


def _sc_body(edge_hbm, table_hbm, out_hbm,
             sidx, didx, r0, r1, acc, s0, s1, isem):
    c = lax.axis_index("c")
    s = lax.axis_index("s")
    w = c * NS + s
    rows = (r0, r1)
    sems = (s0, s1)

    zero = jnp.zeros((16,), jnp.float32)

    @pl.loop(0, C)
    def _(i):
        @pl.loop(0, D // 16)
        def _(k):
            r0[i, pl.ds(k * 16, 16)] = zero

    for r in range(4):
        pltpu.async_copy(r0, acc.at[pl.ds(s * RPT + r * C, C)], s0)
    pltpu.async_copy(r0.at[pl.ds(0, 120)],
                     acc.at[pl.ds(s * RPT + 4 * C, 120)], s0)
    for r in range(4):
        pltpu.make_async_copy(r0, acc.at[pl.ds(s * RPT, C)], s0).wait()
    pltpu.make_async_copy(r0.at[pl.ds(0, 120)],
                          acc.at[pl.ds(s * RPT, 120)], s0).wait()
    plsc.subcore_barrier()

    src_all = edge_hbm.at[0]
    dst_all = edge_hbm.at[1]
    pltpu.sync_copy(src_all.at[w, 0], sidx.at[0])
    pltpu.sync_copy(dst_all.at[w, 0], didx.at[0])

    @pl.loop(0, NBLK)
    def _(b):
        q = lax.rem(b, 2)
        sq, dq = sidx.at[q], didx.at[q]

        def start_g(t, p):
            pltpu.async_copy(table_hbm.at[sq.at[t]], rows[p], sems[p])

        def wait_g(t, p):
            pltpu.make_async_copy(table_hbm.at[sq.at[t]], rows[p],
                                  sems[p]).wait()

        def start_s(t, p):
            pltpu.async_copy(rows[p], acc.at[dq.at[t]], sems[p], add=True)

        def wait_s(t, p):
            pltpu.make_async_copy(rows[p], acc.at[dq.at[t]], sems[p]).wait()

        @pl.when(b + 1 < NBLK)
        def _():
            pltpu.async_copy(src_all.at[w, b + 1], sidx.at[1 - q], isem)
            pltpu.async_copy(dst_all.at[w, b + 1], didx.at[1 - q], isem)

        start_g(0, 0)
        start_g(1, 1)
        wait_g(0, 0); start_s(0, 0)
        for t in range(1, IB - 1):
            p, o = t % 2, (t + 1) % 2
            wait_s(t - 1, o)
            start_g(t + 1, o)
            wait_g(t, p)
            start_s(t, p)
        wait_g(IB - 1, 1); start_s(IB - 1, 1)
        wait_s(IB - 2, 0)
        wait_s(IB - 1, 1)

        @pl.when(b + 1 < NBLK)
        def _():
            pltpu.make_async_copy(src_all.at[w, b + 1], sidx.at[1 - q],
                                  isem).wait()
            pltpu.make_async_copy(dst_all.at[w, b + 1], didx.at[1 - q],
                                  isem).wait()

    plsc.subcore_barrier()

    for r in range(4):
        pltpu.async_copy(acc.at[pl.ds(s * RPT + r * C, C)],
                         out_hbm.at[c].at[pl.ds(s * RPT + r * C, C)],
                         sems[r % NBUF])
    pltpu.async_copy(acc.at[pl.ds(s * RPT + 4 * C, 120)],
                     out_hbm.at[c].at[pl.ds(s * RPT + 4 * C, 120)], isem)
    for r in range(4):
        pltpu.make_async_copy(acc.at[pl.ds(s * RPT, C)],
                              out_hbm.at[c].at[pl.ds(s * RPT, C)],
                              sems[r % NBUF]).wait()
    pltpu.make_async_copy(acc.at[pl.ds(s * RPT, 120)],
                          out_hbm.at[c].at[pl.ds(s * RPT, 120)], isem).wait()


@functools.cache
def _sc_call():
    return pl.kernel(
        _sc_body,
        out_type=jax.ShapeDtypeStruct((NC, AR, D), jnp.float32),
        mesh=plsc.VectorSubcoreMesh(core_axis_name="c", subcore_axis_name="s",
                                    num_cores=NC, num_subcores=NS),
        scratch_types=[
            pltpu.VMEM((2, IB, C), jnp.int32),
            pltpu.VMEM((2, IB, C), jnp.int32),
            pltpu.VMEM((C, D), jnp.float32),
            pltpu.VMEM((C, D), jnp.float32),
            pltpu.VMEM_SHARED((AR, D), jnp.float32),
            pltpu.SemaphoreType.DMA,
            pltpu.SemaphoreType.DMA,
            pltpu.SemaphoreType.DMA,
        ],
    )


def _combine_body(p_ref, o_ref):
    o_ref[...] = p_ref[0] + p_ref[1]


def kernel(i_node, edge_index):
    pad_dst = N_D + jnp.arange(EP - E, dtype=jnp.int32) % (AR - N_D)
    pad = jnp.stack([jnp.zeros(EP - E, jnp.int32), pad_dst])
    edges = jnp.concatenate([edge_index.astype(jnp.int32), pad], axis=1)
    edges = edges.reshape(2, NW, NBLK, IB, C)
    partials = _sc_call()(edges, i_node)
    nb = 10
    rb = N_D // nb
    return pl.pallas_call(
        _combine_body,
        out_shape=jax.ShapeDtypeStruct((N_D, D), jnp.float32),
        grid=(nb,),
        in_specs=[pl.BlockSpec((NC, rb, D), lambda i: (0, i, 0))],
        out_specs=pl.BlockSpec((rb, D), lambda i: (i, 0)),
    )(partials)


# speedup vs baseline: 3.5744x; 3.5744x over previous
import functools

import jax
import jax.numpy as jnp
from jax import lax
from jax.experimental import pallas as pl
from jax.experimental.pallas import tpu as pltpu
from jax.experimental.pallas import tpu_sc as plsc

N_I = 10000
N_D = 10000
E = 320000
D = 128

NC = 2
NS = 16
NW = NC * NS
EP = 327680
C = 128
NBLK = 10
IB = 8
AR = 10112
RPT = AR // NS
NBUF = 2


def _sc_body(edge_hbm, table_hbm, out_hbm,
             sidx, didx, r0, r1, acc, s0, s1, isem):
    c = lax.axis_index("c")
    s = lax.axis_index("s")
    w = c * NS + s
    rows = (r0, r1)
    sems = (s0, s1)

    zero = jnp.zeros((16,), jnp.float32)

    @pl.loop(0, C)
    def _(i):
        @pl.loop(0, D // 16)
        def _(k):
            r0[i, pl.ds(k * 16, 16)] = zero

    for r in range(4):
        pltpu.async_copy(r0, acc.at[pl.ds(s * RPT + r * C, C)], s0)
    pltpu.async_copy(r0.at[pl.ds(0, 120)],
                     acc.at[pl.ds(s * RPT + 4 * C, 120)], s0)
    for r in range(4):
        pltpu.make_async_copy(r0, acc.at[pl.ds(s * RPT, C)], s0).wait()
    pltpu.make_async_copy(r0.at[pl.ds(0, 120)],
                          acc.at[pl.ds(s * RPT, 120)], s0).wait()
    plsc.subcore_barrier()

    src_all = edge_hbm.at[0]
    dst_all = edge_hbm.at[1]
    pltpu.sync_copy(src_all.at[w, 0], sidx.at[0])
    pltpu.sync_copy(dst_all.at[w, 0], didx.at[0])

    @pl.loop(0, NBLK)
    def _(b):
        q = lax.rem(b, 2)
        sq, dq = sidx.at[q], didx.at[q]

        def start_g(t, p):
            pltpu.async_copy(table_hbm.at[sq.at[t]], rows[p], sems[p])

        def wait_g(t, p):
            pltpu.make_async_copy(table_hbm.at[sq.at[t]], rows[p],
                                  sems[p]).wait()

        def start_s(t, p):
            pltpu.async_copy(rows[p], acc.at[dq.at[t]], sems[p], add=True)

        def wait_s(t, p):
            pltpu.make_async_copy(rows[p], acc.at[dq.at[t]], sems[p]).wait()

        @pl.when(b + 1 < NBLK)
        def _():
            pltpu.async_copy(src_all.at[w, b + 1], sidx.at[1 - q], isem)
            pltpu.async_copy(dst_all.at[w, b + 1], didx.at[1 - q], isem)

        start_g(0, 0)
        start_g(1, 1)
        wait_g(0, 0); start_s(0, 0)
        for t in range(1, IB - 1):
            p, o = t % 2, (t + 1) % 2
            wait_s(t - 1, o)
            start_g(t + 1, o)
            wait_g(t, p)
            start_s(t, p)
        wait_g(IB - 1, 1); start_s(IB - 1, 1)
        wait_s(IB - 2, 0)
        wait_s(IB - 1, 1)

        @pl.when(b + 1 < NBLK)
        def _():
            pltpu.make_async_copy(src_all.at[w, b + 1], sidx.at[1 - q],
                                  isem).wait()
            pltpu.make_async_copy(dst_all.at[w, b + 1], didx.at[1 - q],
                                  isem).wait()

    plsc.subcore_barrier()

    for r in range(4):
        pltpu.async_copy(acc.at[pl.ds(s * RPT + r * C, C)],
                         out_hbm.at[c].at[pl.ds(s * RPT + r * C, C)],
                         sems[r % NBUF])
    pltpu.async_copy(acc.at[pl.ds(s * RPT + 4 * C, 120)],
                     out_hbm.at[c].at[pl.ds(s * RPT + 4 * C, 120)], isem)
    for r in range(4):
        pltpu.make_async_copy(acc.at[pl.ds(s * RPT, C)],
                              out_hbm.at[c].at[pl.ds(s * RPT, C)],
                              sems[r % NBUF]).wait()
    pltpu.make_async_copy(acc.at[pl.ds(s * RPT, 120)],
                          out_hbm.at[c].at[pl.ds(s * RPT, 120)], isem).wait()


@functools.cache
def _sc_call():
    return pl.kernel(
        _sc_body,
        out_type=jax.ShapeDtypeStruct((NC, AR, D), jnp.float32),
        mesh=plsc.VectorSubcoreMesh(core_axis_name="c", subcore_axis_name="s",
                                    num_cores=NC, num_subcores=NS),
        scratch_types=[
            pltpu.VMEM((2, IB, C), jnp.int32),
            pltpu.VMEM((2, IB, C), jnp.int32),
            pltpu.VMEM((C, D), jnp.float32),
            pltpu.VMEM((C, D), jnp.float32),
            pltpu.VMEM_SHARED((AR, D), jnp.float32),
            pltpu.SemaphoreType.DMA,
            pltpu.SemaphoreType.DMA,
            pltpu.SemaphoreType.DMA,
        ],
    )


def _combine_body(p_ref, o_ref):
    o_ref[...] = p_ref[0] + p_ref[1]


def kernel(i_node, edge_index):
    pad_iota = jnp.arange(EP - E, dtype=jnp.int32)
    pad = jnp.stack([pad_iota % N_I, N_D + pad_iota % (AR - N_D)])
    edges = jnp.concatenate([edge_index.astype(jnp.int32), pad], axis=1)
    edges = edges.reshape(2, NW, NBLK, IB, C)
    partials = _sc_call()(edges, i_node)
    nb = 10
    rb = N_D // nb
    return pl.pallas_call(
        _combine_body,
        out_shape=jax.ShapeDtypeStruct((N_D, D), jnp.float32),
        grid=(nb,),
        in_specs=[pl.BlockSpec((NC, rb, D), lambda i: (0, i, 0))],
        out_specs=pl.BlockSpec((rb, D), lambda i: (i, 0)),
    )(partials)


# speedup vs baseline: 4.1018x; 1.1476x over previous
import functools

import jax
import jax.numpy as jnp
from jax import lax
from jax.experimental import pallas as pl
from jax.experimental.pallas import tpu as pltpu
from jax.experimental.pallas import tpu_sc as plsc

N_I = 10000
N_D = 10000
E = 320000
D = 128

NC = 2
NS = 16
NW = NC * NS
EPW = E // NW
C = 50
NBLK = 5
IB = 40
BE = IB * C
AR = 10240
RPT = AR // NS
ZB = 40
ZR = 128
NBUF = 4


def _sc_body(edge_hbm, table_hbm, out_hbm,
             sidx, didx, r0, r1, r2, r3, acc, s0, s1, s2, s3, isem):
    c = lax.axis_index("c")
    s = lax.axis_index("s")
    w = c * NS + s
    rows = (r0, r1, r2, r3)
    sems = (s0, s1, s2, s3)

    src_all = edge_hbm.at[0]
    dst_all = edge_hbm.at[1]

    def mk(page):
        sq, dq = sidx.at[page], didx.at[page]

        def start_g(t, p):
            pltpu.async_copy(table_hbm.at[sq.at[t]], rows[p], sems[p])

        def wait_g(t, p):
            pltpu.make_async_copy(table_hbm.at[sq.at[t]], rows[p],
                                  sems[p]).wait()

        def start_s(t, p):
            pltpu.async_copy(rows[p], acc.at[dq.at[t]], sems[p], add=True)

        def wait_s(t, p):
            pltpu.make_async_copy(rows[p], acc.at[dq.at[t]], sems[p]).wait()

        return start_g, wait_g, start_s, wait_s

    pltpu.sync_copy(src_all.at[w, 0], sidx.at[0])
    pltpu.sync_copy(dst_all.at[w, 0], didx.at[0])
    g0_start_g, _, _, _ = mk(0)
    g0_start_g(0, 0)
    g0_start_g(1, 1)
    g0_start_g(2, 2)

    zero = jnp.zeros((16,), jnp.float32)

    @pl.loop(0, ZB)
    def _(i):
        @pl.loop(0, D // 16)
        def _(k):
            r3[i, pl.ds(k * 16, 16)] = zero

    zsrc = r3.at[pl.ds(0, ZB)]
    for r in range(RPT // ZB):
        pltpu.async_copy(zsrc, acc.at[pl.ds(s * RPT + r * ZB, ZB)], isem)
    for r in range(RPT // ZB):
        pltpu.make_async_copy(zsrc, acc.at[pl.ds(s * RPT, ZB)], isem).wait()
    plsc.subcore_barrier()

    for b in range(NBLK):
        q = b % 2
        start_g, wait_g, start_s, wait_s = mk(q)
        start_gn, _, _, _ = mk(1 - q)

        if b + 1 < NBLK:
            pltpu.async_copy(src_all.at[w, b + 1], sidx.at[1 - q], isem)
            pltpu.async_copy(dst_all.at[w, b + 1], didx.at[1 - q], isem)

        if b == 0:
            wait_g(0, 0); start_s(0, 0); start_g(3, 3)
            wait_s(0, 0); start_g(4, 0); wait_g(1, 1); start_s(1, 1)
            wait_s(1, 1); start_g(5, 1); wait_g(2, 2); start_s(2, 2)
            wait_s(2, 2); start_g(6, 2); wait_g(3, 3); start_s(3, 3)
            lo = 4
        else:
            lo = 0

        @pl.loop(lo, IB - 4, step=NBUF)
        def _(g):
            for p in range(NBUF):
                t = g + p
                qq = (p + 3) % NBUF
                wait_s(jnp.maximum(t - 1, 0), qq)
                start_g(t + 3, qq)
                wait_g(t, p)
                start_s(t, p)

        if b + 1 < NBLK:
            pltpu.make_async_copy(src_all.at[w, b + 1], sidx.at[1 - q],
                                  isem).wait()
            pltpu.make_async_copy(dst_all.at[w, b + 1], didx.at[1 - q],
                                  isem).wait()
        wait_s(IB - 5, 3); start_g(IB - 1, 3); wait_g(IB - 4, 0); start_s(IB - 4, 0)
        if b + 1 < NBLK:
            wait_s(IB - 4, 0); start_gn(0, 0); wait_g(IB - 3, 1); start_s(IB - 3, 1)
            wait_s(IB - 3, 1); start_gn(1, 1); wait_g(IB - 2, 2); start_s(IB - 2, 2)
            wait_s(IB - 2, 2); start_gn(2, 2); wait_g(IB - 1, 3); start_s(IB - 1, 3)
        else:
            wait_s(IB - 4, 0); wait_g(IB - 3, 1); start_s(IB - 3, 1)
            wait_s(IB - 3, 1); wait_g(IB - 2, 2); start_s(IB - 2, 2)
            wait_s(IB - 2, 2); wait_g(IB - 1, 3); start_s(IB - 1, 3)
            wait_s(IB - 1, 3)

    plsc.subcore_barrier()

    for r in range(RPT // ZR):
        base = s * RPT + r * ZR
        pltpu.async_copy(acc.at[pl.ds(base, ZR)],
                         out_hbm.at[c].at[pl.ds(base, ZR)], sems[r % NBUF])
    for r in range(RPT // ZR):
        base = s * RPT + r * ZR
        pltpu.make_async_copy(acc.at[pl.ds(base, ZR)],
                              out_hbm.at[c].at[pl.ds(base, ZR)],
                              sems[r % NBUF]).wait()


@functools.cache
def _sc_call():
    return pl.kernel(
        _sc_body,
        out_type=jax.ShapeDtypeStruct((NC, AR, D), jnp.float32),
        mesh=plsc.VectorSubcoreMesh(core_axis_name="c", subcore_axis_name="s",
                                    num_cores=NC, num_subcores=NS),
        scratch_types=[
            pltpu.VMEM((2, IB, C), jnp.int32),
            pltpu.VMEM((2, IB, C), jnp.int32),
            pltpu.VMEM((C, D), jnp.float32),
            pltpu.VMEM((C, D), jnp.float32),
            pltpu.VMEM((C, D), jnp.float32),
            pltpu.VMEM((C, D), jnp.float32),
            pltpu.VMEM_SHARED((AR, D), jnp.float32),
            pltpu.SemaphoreType.DMA,
            pltpu.SemaphoreType.DMA,
            pltpu.SemaphoreType.DMA,
            pltpu.SemaphoreType.DMA,
            pltpu.SemaphoreType.DMA,
        ],
    )


def _combine_body(p_ref, o_ref):
    o_ref[...] = p_ref[0] + p_ref[1]


def kernel(i_node, edge_index):
    edges = edge_index.astype(jnp.int32).reshape(2, NW, NBLK, IB, C)
    partials = _sc_call()(edges, i_node)
    nb = 10
    rb = N_D // nb
    return pl.pallas_call(
        _combine_body,
        out_shape=jax.ShapeDtypeStruct((N_D, D), jnp.float32),
        grid=(nb,),
        in_specs=[pl.BlockSpec((NC, rb, D), lambda i: (0, i, 0))],
        out_specs=pl.BlockSpec((rb, D), lambda i: (i, 0)),
    )(partials)


# speedup vs baseline: 4.1611x; 1.0144x over previous
import functools

import jax
import jax.numpy as jnp
from jax import lax
from jax.experimental import pallas as pl
from jax.experimental.pallas import tpu as pltpu
from jax.experimental.pallas import tpu_sc as plsc

N_I = 10000
N_D = 10000
E = 320000
D = 128

NC = 2
NS = 16
NW = NC * NS
EPW = E // NW
C = 50
NBLK = 5
IB = 40
BE = IB * C
AR = 10240
RPT = AR // NS
ZB = 40
ZR = 128
NBUF = 4


def _sc_body(edge_hbm, table_hbm, out_hbm,
             sidx, didx, r0, r1, r2, r3, acc, s0, s1, s2, s3, isem):
    c = lax.axis_index("c")
    s = lax.axis_index("s")
    w = c * NS + s
    rows = (r0, r1, r2, r3)
    sems = (s0, s1, s2, s3)

    src_all = edge_hbm.at[0]
    dst_all = edge_hbm.at[1]

    def mk(page):
        sq, dq = sidx.at[page], didx.at[page]

        def start_g(t, p):
            pltpu.async_copy(table_hbm.at[sq.at[t]], rows[p], sems[p])

        def wait_g(t, p):
            pltpu.make_async_copy(table_hbm.at[sq.at[t]], rows[p],
                                  sems[p]).wait()

        def start_s(t, p):
            pltpu.async_copy(rows[p], acc.at[dq.at[t]], sems[p], add=True)

        def wait_s(t, p):
            pltpu.make_async_copy(rows[p], acc.at[dq.at[t]], sems[p]).wait()

        return start_g, wait_g, start_s, wait_s

    pltpu.sync_copy(src_all.at[w, 0], sidx.at[0])
    pltpu.sync_copy(dst_all.at[w, 0], didx.at[0])
    g0_start_g, _, _, _ = mk(0)
    g0_start_g(0, 0)
    g0_start_g(1, 1)
    g0_start_g(2, 2)

    zero = jnp.zeros((16,), jnp.float32)

    @pl.loop(0, ZB)
    def _(i):
        @pl.loop(0, D // 16)
        def _(k):
            r3[i, pl.ds(k * 16, 16)] = zero

    zsrc = r3.at[pl.ds(0, ZB)]
    for r in range(RPT // ZB):
        pltpu.async_copy(zsrc, acc.at[pl.ds(s * RPT + r * ZB, ZB)], isem)
    for r in range(RPT // ZB):
        pltpu.make_async_copy(zsrc, acc.at[pl.ds(s * RPT, ZB)], isem).wait()
    plsc.subcore_barrier()

    for b in range(NBLK):
        q = b % 2
        start_g, wait_g, start_s, wait_s = mk(q)
        start_gn, _, _, _ = mk(1 - q)

        if b + 1 < NBLK:
            pltpu.async_copy(src_all.at[w, b + 1], sidx.at[1 - q], isem)
            pltpu.async_copy(dst_all.at[w, b + 1], didx.at[1 - q], isem)

        if b == 0:
            wait_g(0, 0); start_s(0, 0); start_g(3, 3)
            wait_s(0, 0); start_g(4, 0); wait_g(1, 1); start_s(1, 1)
            wait_s(1, 1); start_g(5, 1); wait_g(2, 2); start_s(2, 2)
            wait_s(2, 2); start_g(6, 2); wait_g(3, 3); start_s(3, 3)
            lo = 4
        else:
            lo = 0

        @pl.loop(lo, IB - 4, step=NBUF)
        def _(g):
            for p in range(NBUF):
                t = g + p
                qq = (p + 3) % NBUF
                wait_s(jnp.maximum(t - 1, 0), qq)
                start_g(t + 3, qq)
                wait_g(t, p)
                start_s(t, p)

        if b + 1 < NBLK:
            pltpu.make_async_copy(src_all.at[w, b + 1], sidx.at[1 - q],
                                  isem).wait()
            pltpu.make_async_copy(dst_all.at[w, b + 1], didx.at[1 - q],
                                  isem).wait()
        wait_s(IB - 5, 3); start_g(IB - 1, 3); wait_g(IB - 4, 0); start_s(IB - 4, 0)
        if b + 1 < NBLK:
            wait_s(IB - 4, 0); start_gn(0, 0); wait_g(IB - 3, 1); start_s(IB - 3, 1)
            wait_s(IB - 3, 1); start_gn(1, 1); wait_g(IB - 2, 2); start_s(IB - 2, 2)
            wait_s(IB - 2, 2); start_gn(2, 2); wait_g(IB - 1, 3); start_s(IB - 1, 3)
        else:
            wait_s(IB - 4, 0); wait_g(IB - 3, 1); start_s(IB - 3, 1)
            wait_s(IB - 3, 1); wait_g(IB - 2, 2); start_s(IB - 2, 2)
            wait_s(IB - 2, 2); wait_g(IB - 1, 3); start_s(IB - 1, 3)
            wait_s(IB - 1, 3)

    plsc.subcore_barrier()

    for r in range(RPT // ZR):
        base = s * RPT + r * ZR
        pltpu.async_copy(acc.at[pl.ds(base, ZR)],
                         out_hbm.at[c].at[pl.ds(base, ZR)], sems[r % NBUF])
    for r in range(RPT // ZR):
        base = s * RPT + r * ZR
        pltpu.make_async_copy(acc.at[pl.ds(base, ZR)],
                              out_hbm.at[c].at[pl.ds(base, ZR)],
                              sems[r % NBUF]).wait()


@functools.cache
def _sc_call():
    return pl.kernel(
        _sc_body,
        out_type=jax.ShapeDtypeStruct((NC, AR, D), jnp.float32),
        mesh=plsc.VectorSubcoreMesh(core_axis_name="c", subcore_axis_name="s",
                                    num_cores=NC, num_subcores=NS),
        scratch_types=[
            pltpu.VMEM((2, IB, C), jnp.int32),
            pltpu.VMEM((2, IB, C), jnp.int32),
            pltpu.VMEM((C, D), jnp.float32),
            pltpu.VMEM((C, D), jnp.float32),
            pltpu.VMEM((C, D), jnp.float32),
            pltpu.VMEM((C, D), jnp.float32),
            pltpu.VMEM_SHARED((AR, D), jnp.float32),
            pltpu.SemaphoreType.DMA,
            pltpu.SemaphoreType.DMA,
            pltpu.SemaphoreType.DMA,
            pltpu.SemaphoreType.DMA,
            pltpu.SemaphoreType.DMA,
        ],
    )


def _combine_body(p_ref, o_ref):
    o_ref[...] = p_ref[0] + p_ref[1]


def kernel(i_node, edge_index):
    edges = edge_index.astype(jnp.int32).reshape(2, NW, NBLK, IB, C)
    partials = _sc_call()(edges, i_node)
    nb = 5
    rb = N_D // nb
    return pl.pallas_call(
        _combine_body,
        out_shape=jax.ShapeDtypeStruct((N_D, D), jnp.float32),
        grid=(nb,),
        in_specs=[pl.BlockSpec((NC, rb, D), lambda i: (0, i, 0))],
        out_specs=pl.BlockSpec((rb, D), lambda i: (i, 0)),
    )(partials)


# speedup vs baseline: 4.2235x; 1.0150x over previous
import functools

import jax
import jax.numpy as jnp
from jax import lax
from jax.experimental import pallas as pl
from jax.experimental.pallas import tpu as pltpu
from jax.experimental.pallas import tpu_sc as plsc

N_I = 10000
N_D = 10000
E = 320000
D = 128

NC = 2
NS = 16
NW = NC * NS
EPW = E // NW
C = 50
NBLK = 5
IB = 40
BE = IB * C
AR = 10240
RPT = AR // NS
ZB = 40
ZR = 128
NBUF = 4


def _sc_body(edge_hbm, table_hbm, out_hbm,
             sidx, didx, r0, r1, r2, r3, acc, s0, s1, s2, s3, isem):
    c = lax.axis_index("c")
    s = lax.axis_index("s")
    w = c * NS + s
    rows = (r0, r1, r2, r3)
    sems = (s0, s1, s2, s3)

    src_all = edge_hbm.at[0]
    dst_all = edge_hbm.at[1]

    def mk(page):
        sq, dq = sidx.at[page], didx.at[page]

        def start_g(t, p):
            pltpu.async_copy(table_hbm.at[sq.at[t]], rows[p], sems[p])

        def wait_g(t, p):
            pltpu.make_async_copy(table_hbm.at[sq.at[t]], rows[p],
                                  sems[p]).wait()

        def start_s(t, p):
            pltpu.async_copy(rows[p], acc.at[dq.at[t]], sems[p], add=True)

        def wait_s(t, p):
            pltpu.make_async_copy(rows[p], acc.at[dq.at[t]], sems[p]).wait()

        return start_g, wait_g, start_s, wait_s

    pltpu.sync_copy(src_all.at[w, 0], sidx.at[0])
    pltpu.sync_copy(dst_all.at[w, 0], didx.at[0])
    g0_start_g, _, _, _ = mk(0)
    g0_start_g(0, 0)
    g0_start_g(1, 1)
    g0_start_g(2, 2)

    zero = jnp.zeros((16,), jnp.float32)

    @pl.loop(0, ZB)
    def _(i):
        @pl.loop(0, D // 16)
        def _(k):
            r3[i, pl.ds(k * 16, 16)] = zero

    zsrc = r3.at[pl.ds(0, ZB)]
    for r in range(RPT // ZB):
        pltpu.async_copy(zsrc, acc.at[pl.ds(s * RPT + r * ZB, ZB)], isem)
    for r in range(RPT // ZB):
        pltpu.make_async_copy(zsrc, acc.at[pl.ds(s * RPT, ZB)], isem).wait()
    plsc.subcore_barrier()

    for b in range(NBLK):
        q = b % 2
        start_g, wait_g, start_s, wait_s = mk(q)
        start_gn, _, _, _ = mk(1 - q)

        if b + 1 < NBLK:
            pltpu.async_copy(src_all.at[w, b + 1], sidx.at[1 - q], isem)
            pltpu.async_copy(dst_all.at[w, b + 1], didx.at[1 - q], isem)

        if b == 0:
            wait_g(0, 0); start_s(0, 0); start_g(3, 3)
            wait_s(0, 0); start_g(4, 0); wait_g(1, 1); start_s(1, 1)
            wait_s(1, 1); start_g(5, 1); wait_g(2, 2); start_s(2, 2)
            wait_s(2, 2); start_g(6, 2); wait_g(3, 3); start_s(3, 3)
            lo = 4
        else:
            lo = 0

        @pl.loop(lo, IB - 4, step=NBUF)
        def _(g):
            for p in range(NBUF):
                t = g + p
                qq = (p + 3) % NBUF
                wait_s(jnp.maximum(t - 1, 0), qq)
                start_g(t + 3, qq)
                wait_g(t, p)
                start_s(t, p)

        if b + 1 < NBLK:
            pltpu.make_async_copy(src_all.at[w, b + 1], sidx.at[1 - q],
                                  isem).wait()
            pltpu.make_async_copy(dst_all.at[w, b + 1], didx.at[1 - q],
                                  isem).wait()
        wait_s(IB - 5, 3); start_g(IB - 1, 3); wait_g(IB - 4, 0); start_s(IB - 4, 0)
        if b + 1 < NBLK:
            wait_s(IB - 4, 0); start_gn(0, 0); wait_g(IB - 3, 1); start_s(IB - 3, 1)
            wait_s(IB - 3, 1); start_gn(1, 1); wait_g(IB - 2, 2); start_s(IB - 2, 2)
            wait_s(IB - 2, 2); start_gn(2, 2); wait_g(IB - 1, 3); start_s(IB - 1, 3)
        else:
            wait_s(IB - 4, 0); wait_g(IB - 3, 1); start_s(IB - 3, 1)
            wait_s(IB - 3, 1); wait_g(IB - 2, 2); start_s(IB - 2, 2)
            wait_s(IB - 2, 2); wait_g(IB - 1, 3); start_s(IB - 1, 3)
            wait_s(IB - 1, 3)

    plsc.subcore_barrier()

    for r in range(RPT // ZR):
        base = s * RPT + r * ZR
        pltpu.async_copy(acc.at[pl.ds(base, ZR)],
                         out_hbm.at[c].at[pl.ds(base, ZR)], sems[r % NBUF])
    for r in range(RPT // ZR):
        base = s * RPT + r * ZR
        pltpu.make_async_copy(acc.at[pl.ds(base, ZR)],
                              out_hbm.at[c].at[pl.ds(base, ZR)],
                              sems[r % NBUF]).wait()


@functools.cache
def _sc_call():
    return pl.kernel(
        _sc_body,
        out_type=jax.ShapeDtypeStruct((NC, AR, D), jnp.float32),
        mesh=plsc.VectorSubcoreMesh(core_axis_name="c", subcore_axis_name="s",
                                    num_cores=NC, num_subcores=NS),
        scratch_types=[
            pltpu.VMEM((2, IB, C), jnp.int32),
            pltpu.VMEM((2, IB, C), jnp.int32),
            pltpu.VMEM((C, D), jnp.float32),
            pltpu.VMEM((C, D), jnp.float32),
            pltpu.VMEM((C, D), jnp.float32),
            pltpu.VMEM((C, D), jnp.float32),
            pltpu.VMEM_SHARED((AR, D), jnp.float32),
            pltpu.SemaphoreType.DMA,
            pltpu.SemaphoreType.DMA,
            pltpu.SemaphoreType.DMA,
            pltpu.SemaphoreType.DMA,
            pltpu.SemaphoreType.DMA,
        ],
    )


def _combine_body(p_ref, o_ref):
    o_ref[...] = p_ref[0] + p_ref[1]


def kernel(i_node, edge_index):
    edges = edge_index.astype(jnp.int32).reshape(2, NW, NBLK, IB, C)
    partials = _sc_call()(edges, i_node)
    nb = 2
    rb = N_D // nb
    return pl.pallas_call(
        _combine_body,
        out_shape=jax.ShapeDtypeStruct((N_D, D), jnp.float32),
        grid=(nb,),
        in_specs=[pl.BlockSpec((NC, rb, D), lambda i: (0, i, 0))],
        out_specs=pl.BlockSpec((rb, D), lambda i: (i, 0)),
    )(partials)
